# baseline (device time: 30130 ns/iter reference)
import jax
import jax.numpy as jnp
from jax import lax
from jax.experimental import pallas as pl
from jax.experimental.pallas import tpu as pltpu

N_DEV = 4
N_LAYERS = 3
B = 512
D = 256
M = B // N_DEV


def kernel(x, Win0, Wout0, Win1, Wout1, Win2, Wout2):
    def body(x_ref, win0, wout0, win1, wout1, win2, wout2,
             out_ref, part_ref, rs_ref, xg_ref,
             rs_send_sems, rs_recv_sems, ag_send_sems, ag_recv_sems):
        my = lax.axis_index("i")

        barrier_sem = pltpu.get_barrier_semaphore()
        for d in range(1, N_DEV):
            pl.semaphore_signal(
                barrier_sem, inc=1,
                device_id=(lax.rem(my + d, N_DEV),),
                device_id_type=pl.DeviceIdType.MESH,
            )

        pending_sends = []
        wins = [win0, win1, win2]
        wouts = [wout0, wout1, wout2]
        for k in range(N_LAYERS):
            xb = x_ref[...] .astype(jnp.bfloat16) if k == 0 else xg_ref[k - 1]
            h = jnp.dot(xb, wins[k][...].astype(jnp.bfloat16),
                        preferred_element_type=jnp.float32)
            h = jnp.maximum(h, 0.0).astype(jnp.bfloat16)
            partial = jnp.dot(h, wouts[k][...].astype(jnp.bfloat16),
                              preferred_element_type=jnp.float32)
            part_ref[k, :, :] = partial.astype(jnp.bfloat16)

            if k == 0:
                pl.semaphore_wait(barrier_sem, N_DEV - 1)

            rs_rdmas = []
            for d in range(1, N_DEV):
                t = lax.rem(my + d, N_DEV)
                rdma = pltpu.make_async_remote_copy(
                    src_ref=part_ref.at[k].at[pl.ds(t * M, M)],
                    dst_ref=rs_ref.at[k, d - 1],
                    send_sem=rs_send_sems.at[k, d - 1],
                    recv_sem=rs_recv_sems.at[k, d - 1],
                    device_id=(t,),
                    device_id_type=pl.DeviceIdType.MESH,
                )
                rdma.start()
                rs_rdmas.append(rdma)
            for rdma in rs_rdmas:
                rdma.wait_recv()
            pending_sends.extend(rs_rdmas)

            red = part_ref[k, pl.ds(my * M, M), :].astype(jnp.float32)
            for j in range(N_DEV - 1):
                red = red + rs_ref[k, j].astype(jnp.float32)

            if k < N_LAYERS - 1:
                xg_ref[k, pl.ds(my * M, M), :] = red.astype(jnp.bfloat16)
                ag_rdmas = []
                for d in range(1, N_DEV):
                    rdma = pltpu.make_async_remote_copy(
                        src_ref=xg_ref.at[k].at[pl.ds(my * M, M)],
                        dst_ref=xg_ref.at[k].at[pl.ds(my * M, M)],
                        send_sem=ag_send_sems.at[k, d - 1],
                        recv_sem=ag_recv_sems.at[k, d - 1],
                        device_id=(lax.rem(my + d, N_DEV),),
                        device_id_type=pl.DeviceIdType.MESH,
                    )
                    rdma.start()
                    ag_rdmas.append(rdma)
                for rdma in ag_rdmas:
                    rdma.wait_recv()
                pending_sends.extend(ag_rdmas)
            else:
                out_ref[...] = red

        for rdma in pending_sends:
            rdma.wait_send()

    return pl.pallas_call(
        body,
        out_shape=jax.ShapeDtypeStruct((M, D), jnp.float32),
        in_specs=[pl.BlockSpec(memory_space=pltpu.VMEM)] * 7,
        out_specs=pl.BlockSpec(memory_space=pltpu.VMEM),
        scratch_shapes=[
            pltpu.VMEM((N_LAYERS, B, D), jnp.bfloat16),
            pltpu.VMEM((N_LAYERS, N_DEV - 1, M, D), jnp.bfloat16),
            pltpu.VMEM((N_LAYERS - 1, B, D), jnp.bfloat16),
            pltpu.SemaphoreType.DMA((N_LAYERS, N_DEV - 1)),
            pltpu.SemaphoreType.DMA((N_LAYERS, N_DEV - 1)),
            pltpu.SemaphoreType.DMA((N_LAYERS - 1, N_DEV - 1)),
            pltpu.SemaphoreType.DMA((N_LAYERS - 1, N_DEV - 1)),
        ],
        compiler_params=pltpu.CompilerParams(collective_id=0),
    )(x, Win0, Wout0, Win1, Wout1, Win2, Wout2)


# device time: 29634 ns/iter; 1.0167x vs baseline; 1.0167x over previous
import jax
import jax.numpy as jnp
from jax import lax
from jax.experimental import pallas as pl
from jax.experimental.pallas import tpu as pltpu

N_DEV = 4
N_LAYERS = 3
B = 512
D = 256
M = B // N_DEV

_D_ORDER = (1, 3, 2)


def kernel(x, Win0, Wout0, Win1, Wout1, Win2, Wout2):
    def body(x_ref, win0, wout0, win1, wout1, win2, wout2,
             out_ref, part_ref, rs_ref, xg_ref,
             rs_send_sems, rs_recv_sems, ag_send_sems, ag_recv_sems):
        my = lax.axis_index("i")

        barrier_sem = pltpu.get_barrier_semaphore()
        for d in range(1, N_DEV):
            pl.semaphore_signal(
                barrier_sem, inc=1,
                device_id=(lax.rem(my + d, N_DEV),),
                device_id_type=pl.DeviceIdType.MESH,
            )

        pending_sends = []
        wins = [win0, win1, win2]
        wouts = [wout0, wout1, wout2]
        barrier_done = [False]

        def chunk_partial(k, wi, wo, xt):
            hh = jnp.dot(xt, wi, preferred_element_type=jnp.float32)
            hh = jnp.maximum(hh, 0.0).astype(jnp.bfloat16)
            return jnp.dot(hh, wo, preferred_element_type=jnp.float32)

        def start_rs_send(k, t, dprime):
            rdma = pltpu.make_async_remote_copy(
                src_ref=part_ref.at[k].at[pl.ds(t * M, M)],
                dst_ref=rs_ref.at[k, dprime - 1],
                send_sem=rs_send_sems.at[k, dprime - 1],
                recv_sem=rs_recv_sems.at[k, dprime - 1],
                device_id=(t,),
                device_id_type=pl.DeviceIdType.MESH,
            )
            if not barrier_done[0]:
                pl.semaphore_wait(barrier_sem, N_DEV - 1)
                barrier_done[0] = True
            rdma.start()
            pending_sends.append(rdma)

        red = None
        for k in range(N_LAYERS):
            wi = wins[k][...].astype(jnp.bfloat16)
            wo = wouts[k][...].astype(jnp.bfloat16)

            if k == 0:
                for dprime in _D_ORDER:
                    t = lax.rem(my + dprime, N_DEV)
                    pt = chunk_partial(
                        k, wi, wo, x_ref[pl.ds(t * M, M), :].astype(jnp.bfloat16))
                    part_ref[k, pl.ds(t * M, M), :] = pt.astype(jnp.bfloat16)
                    start_rs_send(k, t, dprime)
                p_own = chunk_partial(
                    k, wi, wo, x_ref[pl.ds(my * M, M), :].astype(jnp.bfloat16))
            else:
                p_own = chunk_partial(k, wi, wo, red.astype(jnp.bfloat16))
                for d in _D_ORDER:
                    t = lax.rem(my - d + N_DEV, N_DEV)
                    ag_recv = pltpu.make_async_remote_copy(
                        src_ref=xg_ref.at[k - 1].at[pl.ds(t * M, M)],
                        dst_ref=xg_ref.at[k - 1].at[pl.ds(t * M, M)],
                        send_sem=ag_send_sems.at[k - 1, d - 1],
                        recv_sem=ag_recv_sems.at[k - 1, d - 1],
                        device_id=(my,),
                        device_id_type=pl.DeviceIdType.MESH,
                    )
                    ag_recv.wait_recv()
                    pt = chunk_partial(
                        k, wi, wo, xg_ref[k - 1, pl.ds(t * M, M), :])
                    part_ref[k, pl.ds(t * M, M), :] = pt.astype(jnp.bfloat16)
                    dprime = N_DEV - d
                    start_rs_send(k, t, dprime)

            red = p_own
            for d in _D_ORDER:
                rs_rdmas_d = pltpu.make_async_remote_copy(
                    src_ref=part_ref.at[k].at[pl.ds(my * M, M)],
                    dst_ref=rs_ref.at[k, d - 1],
                    send_sem=rs_send_sems.at[k, d - 1],
                    recv_sem=rs_recv_sems.at[k, d - 1],
                    device_id=(my,),
                    device_id_type=pl.DeviceIdType.MESH,
                )
                rs_rdmas_d.wait_recv()
                red = red + rs_ref[k, d - 1].astype(jnp.float32)

            if k < N_LAYERS - 1:
                xg_ref[k, pl.ds(my * M, M), :] = red.astype(jnp.bfloat16)
                ag_rdmas = []
                for d in range(1, N_DEV):
                    rdma = pltpu.make_async_remote_copy(
                        src_ref=xg_ref.at[k].at[pl.ds(my * M, M)],
                        dst_ref=xg_ref.at[k].at[pl.ds(my * M, M)],
                        send_sem=ag_send_sems.at[k, d - 1],
                        recv_sem=ag_recv_sems.at[k, d - 1],
                        device_id=(lax.rem(my + d, N_DEV),),
                        device_id_type=pl.DeviceIdType.MESH,
                    )
                    rdma.start()
                    ag_rdmas.append(rdma)
                pending_sends.extend(ag_rdmas)
            else:
                out_ref[...] = red

        for rdma in pending_sends:
            rdma.wait_send()

    return pl.pallas_call(
        body,
        out_shape=jax.ShapeDtypeStruct((M, D), jnp.float32),
        in_specs=[pl.BlockSpec(memory_space=pltpu.VMEM)] * 7,
        out_specs=pl.BlockSpec(memory_space=pltpu.VMEM),
        scratch_shapes=[
            pltpu.VMEM((N_LAYERS, B, D), jnp.bfloat16),
            pltpu.VMEM((N_LAYERS, N_DEV - 1, M, D), jnp.bfloat16),
            pltpu.VMEM((N_LAYERS - 1, B, D), jnp.bfloat16),
            pltpu.SemaphoreType.DMA((N_LAYERS, N_DEV - 1)),
            pltpu.SemaphoreType.DMA((N_LAYERS, N_DEV - 1)),
            pltpu.SemaphoreType.DMA((N_LAYERS - 1, N_DEV - 1)),
            pltpu.SemaphoreType.DMA((N_LAYERS - 1, N_DEV - 1)),
        ],
        compiler_params=pltpu.CompilerParams(collective_id=0),
    )(x, Win0, Wout0, Win1, Wout1, Win2, Wout2)


# device time: 27887 ns/iter; 1.0804x vs baseline; 1.0626x over previous
import jax
import jax.numpy as jnp
from jax import lax
from jax.experimental import pallas as pl
from jax.experimental.pallas import tpu as pltpu

N_DEV = 4
N_LAYERS = 3
B = 512
D = 256
M = B // N_DEV


def kernel(x, Win0, Wout0, Win1, Wout1, Win2, Wout2):
    def body(x_ref, win0, wout0, win1, wout1, win2, wout2,
             out_ref, part_ref, piece_ref, xin_ref, xout_ref,
             psend_sems, precv_sems, xsend_sems, xrecv_sems):
        my = lax.axis_index("i")
        right = lax.rem(my + 1, N_DEV)
        diag = lax.rem(my + 2, N_DEV)
        left = lax.rem(my + 3, N_DEV)

        barrier_sem = pltpu.get_barrier_semaphore()
        for d in range(1, N_DEV):
            pl.semaphore_signal(
                barrier_sem, inc=1,
                device_id=(lax.rem(my + d, N_DEV),),
                device_id_type=pl.DeviceIdType.MESH,
            )

        pending_sends = []

        def send_piece(k, t_rows, target, slot):
            rdma = pltpu.make_async_remote_copy(
                src_ref=part_ref.at[k].at[pl.ds(t_rows * M, M)],
                dst_ref=piece_ref.at[k, slot],
                send_sem=psend_sems.at[k, slot],
                recv_sem=precv_sems.at[k, slot],
                device_id=(target,),
                device_id_type=pl.DeviceIdType.MESH,
            )
            rdma.start()
            pending_sends.append(rdma)

        def wait_piece(k, slot):
            rdma = pltpu.make_async_remote_copy(
                src_ref=piece_ref.at[k, slot],
                dst_ref=piece_ref.at[k, slot],
                send_sem=psend_sems.at[k, slot],
                recv_sem=precv_sems.at[k, slot],
                device_id=(my,),
                device_id_type=pl.DeviceIdType.MESH,
            )
            rdma.wait_recv()
            return piece_ref[k, slot].astype(jnp.float32)

        def send_x(k, target, slot):
            rdma = pltpu.make_async_remote_copy(
                src_ref=xout_ref.at[k],
                dst_ref=xin_ref.at[k, slot],
                send_sem=xsend_sems.at[k, slot],
                recv_sem=xrecv_sems.at[k, slot],
                device_id=(target,),
                device_id_type=pl.DeviceIdType.MESH,
            )
            rdma.start()
            pending_sends.append(rdma)

        def wait_x(k, slot):
            rdma = pltpu.make_async_remote_copy(
                src_ref=xin_ref.at[k, slot],
                dst_ref=xin_ref.at[k, slot],
                send_sem=xsend_sems.at[k, slot],
                recv_sem=xrecv_sems.at[k, slot],
                device_id=(my,),
                device_id_type=pl.DeviceIdType.MESH,
            )
            rdma.wait_recv()
            return xin_ref[k, slot]

        wins = [win0, win1, win2]
        wouts = [wout0, wout1, wout2]
        red_a = None
        red_b = None

        for k in range(N_LAYERS):
            wi = wins[k][...].astype(jnp.bfloat16)
            wo = wouts[k][...].astype(jnp.bfloat16)
            last = k == N_LAYERS - 1

            def piece(xt):
                hh = jnp.dot(xt, wi, preferred_element_type=jnp.float32)
                hh = jnp.maximum(hh, 0.0).astype(jnp.bfloat16)
                return jnp.dot(hh, wo, preferred_element_type=jnp.float32)

            if k == 0:
                xa = x_ref[pl.ds(my * M, M), :].astype(jnp.bfloat16)
                xb = x_ref[pl.ds(diag * M, M), :].astype(jnp.bfloat16)
            else:
                xa = red_a.astype(jnp.bfloat16)
                xb = red_b.astype(jnp.bfloat16)
            part_ref[k, pl.ds(my * M, M), :] = piece(xa).astype(jnp.bfloat16)
            part_ref[k, pl.ds(diag * M, M), :] = piece(xb).astype(jnp.bfloat16)
            if k == 0:
                pl.semaphore_wait(barrier_sem, N_DEV - 1)
            send_piece(k, diag, diag, 0)
            if not last:
                send_piece(k, my, diag, 1)

            if k == 0:
                xr = x_ref[pl.ds(right * M, M), :].astype(jnp.bfloat16)
            else:
                xr = wait_x(k - 1, 0)
            part_ref[k, pl.ds(right * M, M), :] = piece(xr).astype(jnp.bfloat16)
            send_piece(k, right, right, 4)
            if not last:
                send_piece(k, right, left, 3)

            if k == 0:
                xl = x_ref[pl.ds(left * M, M), :].astype(jnp.bfloat16)
            else:
                xl = wait_x(k - 1, 1)
            part_ref[k, pl.ds(left * M, M), :] = piece(xl).astype(jnp.bfloat16)
            send_piece(k, left, left, 2)
            if not last:
                send_piece(k, left, right, 5)

            red_a = part_ref[k, pl.ds(my * M, M), :].astype(jnp.float32)
            for slot in (2, 4, 0):
                red_a = red_a + wait_piece(k, slot)

            if last:
                out_ref[...] = red_a
            else:
                xout_ref[k, :, :] = red_a.astype(jnp.bfloat16)
                send_x(k, right, 1)
                send_x(k, left, 0)
                red_b = part_ref[k, pl.ds(diag * M, M), :].astype(jnp.float32)
                for slot in (3, 5, 1):
                    red_b = red_b + wait_piece(k, slot)

        for rdma in pending_sends:
            rdma.wait_send()

    return pl.pallas_call(
        body,
        out_shape=jax.ShapeDtypeStruct((M, D), jnp.float32),
        in_specs=[pl.BlockSpec(memory_space=pltpu.VMEM)] * 7,
        out_specs=pl.BlockSpec(memory_space=pltpu.VMEM),
        scratch_shapes=[
            pltpu.VMEM((N_LAYERS, B, D), jnp.bfloat16),
            pltpu.VMEM((N_LAYERS, 6, M, D), jnp.bfloat16),
            pltpu.VMEM((N_LAYERS - 1, 2, M, D), jnp.bfloat16),
            pltpu.VMEM((N_LAYERS - 1, M, D), jnp.bfloat16),
            pltpu.SemaphoreType.DMA((N_LAYERS, 6)),
            pltpu.SemaphoreType.DMA((N_LAYERS, 6)),
            pltpu.SemaphoreType.DMA((N_LAYERS - 1, 2)),
            pltpu.SemaphoreType.DMA((N_LAYERS - 1, 2)),
        ],
        compiler_params=pltpu.CompilerParams(collective_id=0),
    )(x, Win0, Wout0, Win1, Wout1, Win2, Wout2)
